# Initial kernel scaffold; baseline (speedup 1.0000x reference)
import jax
import jax.numpy as jnp
from jax.experimental import pallas as pl


def kernel(Fs, h0_h, h0_c, Ms, Wa, Wh, v, W_ih, b_ih, W_hh, b_hh, W_lt, b_lt):
    raise NotImplementedError("write your pallas kernel here")


# speedup vs baseline: 2.5038x; 2.5038x over previous
import jax
import jax.numpy as jnp
from jax.experimental import pallas as pl
from jax.experimental.pallas import tpu as pltpu

HID = 512
ATTN = 384
VOCAB = 30
B = 8
L = 16
N = 196


def _step(fs_ref, ms_ref, h0h_ref, h0c_ref, wa_ref, wh_ref, vt_ref,
          wih_ref, whh_ref, b2_ref, wlt_ref, blt_ref,
          betas_ref, logits_ref, probs_ref, hout_ref, cout_ref,
          h_s, c_s):
    i = pl.program_id(0)

    @pl.when(i == 0)
    def _():
        h_s[:] = h0h_ref[0]
        c_s[:] = h0c_ref[0]

    fi = fs_ref[0]
    prev = h_s[:]

    fw = jnp.dot(fi, wa_ref[:], preferred_element_type=jnp.float32)
    hw = jnp.dot(prev, wh_ref[:], preferred_element_type=jnp.float32)
    t = jnp.tanh(fw.reshape(N, B, HID) + hw[None, :, :])
    aw = jnp.sum(t * vt_ref[:][None], axis=2)

    m = jnp.max(aw, axis=0, keepdims=True)
    e = jnp.exp(aw - m)
    sm = e / jnp.sum(e, axis=0, keepdims=True)
    awm = sm * ms_ref[0]
    betas_ref[0] = awm

    denom = jnp.clip(jnp.sum(awm, axis=0, keepdims=True), 1e-5, None)
    awn = awm / denom
    s = jnp.sum(awn[:, :, None] * fi.reshape(N, B, ATTN), axis=0)

    gates = (jnp.dot(s, wih_ref[:], preferred_element_type=jnp.float32)
             + jnp.dot(prev, whh_ref[:], preferred_element_type=jnp.float32)
             + b2_ref[:])
    ig = jax.nn.sigmoid(gates[:, :HID])
    fg = jax.nn.sigmoid(gates[:, HID:2 * HID])
    gg = jnp.tanh(gates[:, 2 * HID:3 * HID])
    og = jax.nn.sigmoid(gates[:, 3 * HID:])
    c = fg * c_s[:] + ig * gg
    h = og * jnp.tanh(c)
    h_s[:] = h
    c_s[:] = c
    hout_ref[0] = h
    cout_ref[0] = c

    lg = jnp.dot(h, wlt_ref[:], preferred_element_type=jnp.float32) + blt_ref[:]
    logits_ref[0] = lg
    pm = jnp.max(lg, axis=1, keepdims=True)
    pe = jnp.exp(lg - pm)
    probs_ref[0] = pe / jnp.sum(pe, axis=1, keepdims=True)


def kernel(Fs, h0_h, h0_c, Ms, Wa, Wh, v, W_ih, b_ih, W_hh, b_hh, W_lt, b_lt):
    B_, L_, Fd, hm, wm = Fs.shape
    Fst = jnp.transpose(Fs.reshape(B_, L_, Fd, N), (1, 3, 0, 2)).reshape(L_, N * B_, Fd)
    Msr = jnp.transpose(Ms.reshape(B_, L_, N), (1, 2, 0))
    h0h = jnp.transpose(h0_h, (1, 0, 2))
    h0c = jnp.transpose(h0_c, (1, 0, 2))
    vt = v.T
    wih = W_ih.T
    whh = W_hh.T
    b2 = (b_ih + b_hh)[None, :]
    wlt = W_lt.T
    blt = b_lt[None, :]

    def full(a):
        nd = a.ndim
        return pl.BlockSpec(a.shape, lambda i, _n=nd: (0,) * _n)

    grid = (L_,)
    out_shapes = (
        jax.ShapeDtypeStruct((L_, N, B_), jnp.float32),
        jax.ShapeDtypeStruct((L_, B_, VOCAB), jnp.float32),
        jax.ShapeDtypeStruct((L_, B_, VOCAB), jnp.float32),
        jax.ShapeDtypeStruct((1, B_, HID), jnp.float32),
        jax.ShapeDtypeStruct((1, B_, HID), jnp.float32),
    )
    betas, logits, probs, hx, cx = pl.pallas_call(
        _step,
        grid=grid,
        in_specs=[
            pl.BlockSpec((1, N * B_, Fd), lambda i: (i, 0, 0)),
            pl.BlockSpec((1, N, B_), lambda i: (i, 0, 0)),
            full(h0h), full(h0c), full(Wa), full(Wh), full(vt),
            full(wih), full(whh), full(b2), full(wlt), full(blt),
        ],
        out_specs=(
            pl.BlockSpec((1, N, B_), lambda i: (i, 0, 0)),
            pl.BlockSpec((1, B_, VOCAB), lambda i: (i, 0, 0)),
            pl.BlockSpec((1, B_, VOCAB), lambda i: (i, 0, 0)),
            pl.BlockSpec((1, B_, HID), lambda i: (0, 0, 0)),
            pl.BlockSpec((1, B_, HID), lambda i: (0, 0, 0)),
        ),
        out_shape=out_shapes,
        scratch_shapes=[
            pltpu.VMEM((B_, HID), jnp.float32),
            pltpu.VMEM((B_, HID), jnp.float32),
        ],
        compiler_params=pltpu.CompilerParams(
            dimension_semantics=("arbitrary",),
        ),
    )(Fst, Msr, h0h, h0c, Wa, Wh, vt, wih, whh, b2, wlt, blt)

    logits_o = jnp.transpose(logits, (1, 0, 2))
    probs_o = jnp.transpose(probs, (1, 0, 2))
    betas_o = jnp.transpose(betas, (2, 0, 1)).reshape(B_, L_, hm, wm)
    return logits_o, probs_o, hx, cx, betas_o
